# Initial kernel scaffold; baseline (speedup 1.0000x reference)
import jax
import jax.numpy as jnp
from jax.experimental import pallas as pl


def kernel(rho):
    raise NotImplementedError("write your pallas kernel here")


# speedup vs baseline: 16.4193x; 16.4193x over previous
import jax
import jax.numpy as jnp
from jax.experimental import pallas as pl
from jax.experimental.pallas import tpu as pltpu

_D = 32
_ND = 34


def _insert(a, axis):
    sh = list(a.shape)
    sh[axis] = 1
    z = jnp.zeros(sh, a.dtype)

    def take(lo, hi):
        s = [slice(None)] * a.ndim
        s[axis] = slice(lo, hi)
        return a[tuple(s)]

    return jnp.concatenate([take(0, 5), z, take(5, 15), z, take(15, 32)],
                           axis=axis)


def _body(in_ref, out_ref):
    x = in_ref[0]
    x = x.reshape(_D, _D, _D, _D)
    for ax in range(4):
        x = _insert(x, ax)
    out_ref[0] = x.reshape(_ND * _ND, _ND * _ND)


def kernel(rho):
    b = rho.shape[0]
    return pl.pallas_call(
        _body,
        grid=(b,),
        in_specs=[pl.BlockSpec((1, _D * _D, _D * _D), lambda i: (i, 0, 0))],
        out_specs=pl.BlockSpec((1, _ND * _ND, _ND * _ND), lambda i: (i, 0, 0)),
        out_shape=jax.ShapeDtypeStruct((b, _ND * _ND, _ND * _ND), rho.dtype),
        compiler_params=pltpu.CompilerParams(
            vmem_limit_bytes=100 * 1024 * 1024),
    )(rho)
